# baseline (speedup 1.0000x reference)
import functools

import jax
import jax.numpy as jnp
from jax import lax
from jax.experimental import pallas as pl
from jax.experimental.pallas import tpu as pltpu
from jax.experimental.pallas import tpu_sc as plsc

VOCAB = 100000
DIM = 32
BATCH = 1024

_info = plsc.get_sparse_core_info()
_NC, _NS = _info.num_cores, _info.num_subcores
_NW = _NC * _NS

_sc_mesh = plsc.VectorSubcoreMesh(core_axis_name="c", subcore_axis_name="s")


@functools.partial(
    pl.kernel,
    mesh=_sc_mesh,
    compiler_params=pltpu.CompilerParams(needs_layout_passes=False),
    out_type=jax.ShapeDtypeStruct((DIM, BATCH), jnp.float32),
    scratch_types=[
        pltpu.VMEM((VOCAB,), jnp.float32),
        pltpu.VMEM((BATCH,), jnp.int32),
        pltpu.VMEM((BATCH,), jnp.float32),
        pltpu.SemaphoreType.DMA,
    ],
)
def _sc_gather_t(table_hbm, idx_hbm, out_hbm, row_v, xl_v, out_v, sem):
    f = lax.axis_index("s") * _NC + lax.axis_index("c")
    cp_row = pltpu.async_copy(table_hbm.at[f], row_v, sem)
    pltpu.sync_copy(idx_hbm, xl_v)
    cp_row.wait()
    for g in range(BATCH // 16):
        idx = xl_v[pl.ds(16 * g, 16)]
        out_v[pl.ds(16 * g, 16)] = plsc.load_gather(row_v, [idx])
    pltpu.sync_copy(out_v, out_hbm.at[f])


_BN = 4096
_GRID = (VOCAB + _BN - 1) // _BN


def _proj_body(emb_ref, w_ref, b_ref, out_ref):
    emb_a = jnp.concatenate(
        [emb_ref[...], jnp.ones((1, BATCH), jnp.float32)], axis=0
    )
    w_a = jnp.concatenate([w_ref[...], b_ref[...]], axis=0)
    out_ref[...] = lax.dot_general(
        w_a,
        emb_a,
        dimension_numbers=(((0,), (0,)), ((), ())),
        preferred_element_type=jnp.float32,
    )


def _projection(emb_t, wt, b2):
    return pl.pallas_call(
        _proj_body,
        grid=(_GRID,),
        in_specs=[
            pl.BlockSpec((DIM, BATCH), lambda j: (0, 0)),
            pl.BlockSpec((DIM, _BN), lambda j: (0, j)),
            pl.BlockSpec((1, _BN), lambda j: (0, j)),
        ],
        out_specs=pl.BlockSpec((_BN, BATCH), lambda j: (j, 0)),
        out_shape=jax.ShapeDtypeStruct((VOCAB, BATCH), jnp.float32),
    )(emb_t, wt, b2)


def kernel(x, embed_table, W, b):
    emb_t = _sc_gather_t(embed_table.T, x.astype(jnp.int32))
    out_t = _projection(emb_t, W.T, b.reshape(1, VOCAB))
    return out_t.T


# speedup vs baseline: 1.0057x; 1.0057x over previous
import functools

import jax
import jax.numpy as jnp
from jax import lax
from jax.experimental import pallas as pl
from jax.experimental.pallas import tpu as pltpu
from jax.experimental.pallas import tpu_sc as plsc

VOCAB = 100000
DIM = 32
BATCH = 1024

_info = plsc.get_sparse_core_info()
_NC, _NS = _info.num_cores, _info.num_subcores

_sc_mesh = plsc.VectorSubcoreMesh(core_axis_name="c", subcore_axis_name="s")


@functools.partial(
    pl.kernel,
    mesh=_sc_mesh,
    compiler_params=pltpu.CompilerParams(needs_layout_passes=False),
    out_type=jax.ShapeDtypeStruct((DIM, BATCH), jnp.float32),
    scratch_types=[
        pltpu.VMEM((VOCAB,), jnp.float32),
        pltpu.VMEM((BATCH,), jnp.int32),
        pltpu.VMEM((BATCH,), jnp.float32),
        pltpu.SemaphoreType.DMA,
    ],
)
def _sc_gather_t(table_hbm, idx_hbm, out_hbm, row_v, xl_v, out_v, sem):
    f = lax.axis_index("s") * _NC + lax.axis_index("c")
    cp_row = pltpu.async_copy(table_hbm.at[f], row_v, sem)
    pltpu.sync_copy(idx_hbm, xl_v)
    cp_row.wait()
    for g in range(BATCH // 16):
        idx = xl_v[pl.ds(16 * g, 16)]
        out_v[pl.ds(16 * g, 16)] = plsc.load_gather(row_v, [idx])
    pltpu.sync_copy(out_v, out_hbm.at[f])


_BN = 4096
_GRID = (VOCAB + _BN - 1) // _BN


def _proj_body(emb_ref, w_ref, b_ref, out_ref):
    emb_a = jnp.concatenate(
        [emb_ref[...], jnp.ones((1, BATCH), jnp.float32)], axis=0
    )
    w_a = jnp.concatenate([w_ref[...], b_ref[...]], axis=0)
    out_ref[...] = lax.dot_general(
        w_a,
        emb_a,
        dimension_numbers=(((0,), (0,)), ((), ())),
        preferred_element_type=jnp.float32,
    )


def _projection(emb_t, wt, b2):
    return pl.pallas_call(
        _proj_body,
        grid=(_GRID,),
        in_specs=[
            pl.BlockSpec((DIM, BATCH), lambda j: (0, 0)),
            pl.BlockSpec((DIM, _BN), lambda j: (0, j)),
            pl.BlockSpec((1, _BN), lambda j: (0, j)),
        ],
        out_specs=pl.BlockSpec((_BN, BATCH), lambda j: (j, 0)),
        out_shape=jax.ShapeDtypeStruct((VOCAB, BATCH), jnp.float32),
    )(emb_t, wt, b2)


def kernel(x, embed_table, W, b):
    emb_t = _sc_gather_t(embed_table.T, x.astype(jnp.int32))
    out_t = _projection(emb_t, W.T, b.reshape(1, VOCAB))
    return out_t.T
